# baseline (speedup 1.0000x reference)
import jax
import jax.numpy as jnp
from jax import lax
from jax.experimental import pallas as pl
from jax.experimental.pallas import tpu as pltpu
from jax.experimental.pallas import tpu_sc as plsc

N_MOVIE = 50000
E = 500000
HID = 128
OUT_CH = 16

_BLK = 2048
_NBLK = 25
_BLK_C = 10240
_NBLK_C = 5
_NSLICE = _NBLK * _BLK

_SLAB = 15616
_NCHS = _SLAB // 128
_GRP = 16
_C1_BASE = 249984
_SEG = 3200
_NPAD = 16 * _SEG


def _sc_count_body(e0_hbm, e1_hbm, t0_hbm, t1_hbm, out_hbm, idx_v, ext_v,
                   ones_v, stage_v, sem, acc0_sh, acc1_sh):
    c = lax.axis_index("c")
    s = lax.axis_index("s")
    edges = (e0_hbm, e1_hbm)
    tails = (t0_hbm, t1_hbm)
    accs = (acc0_sh, acc1_sh)

    def fill_ones(j, _):
        ones_v[pl.ds(j * 16, 16)] = jnp.ones((16,), jnp.float32)
        return _
    lax.fori_loop(0, 128 // 16, fill_ones, None)

    def fill_zeros(j, _):
        stage_v[pl.ds(j * 16, 16)] = jnp.zeros((16,), jnp.float32)
        return _
    lax.fori_loop(0, _SEG // 16, fill_zeros, None)

    for t in range(2):
        pltpu.sync_copy(stage_v, accs[t].at[pl.ds(s * _SEG, _SEG)])
    plsc.subcore_barrier()

    base = c * _C1_BASE + s * _SLAB
    for t in range(2):
        acc = accs[t]
        pltpu.sync_copy(edges[t].at[:, pl.ds(base, _SLAB)], idx_v)

        def scatter_grp(jo, _):
            cps = [
                pltpu.async_copy(
                    ones_v,
                    acc.at[idx_v.at[1, pl.ds((jo * _GRP + b) * 128, 128)]],
                    sem, add=True)
                for b in range(_GRP)
            ]
            for cp in cps:
                cp.wait()
            return _
        lax.fori_loop(0, _NCHS // _GRP, scatter_grp, None)
        for j in range(_NCHS - _NCHS % _GRP, _NCHS):
            pltpu.sync_copy(ones_v,
                            acc.at[idx_v.at[1, pl.ds(j * 128, 128)]],
                            add=True)

        @pl.when(jnp.logical_and(c == 0, s == 0))
        def _():
            pltpu.sync_copy(edges[t].at[:, pl.ds(249856, 128)], ext_v)
            pltpu.sync_copy(ones_v, acc.at[ext_v.at[1, :]], add=True)

        @pl.when(jnp.logical_and(c == 1, s == 0))
        def _():
            pltpu.sync_copy(edges[t].at[:, pl.ds(499840, 128)], ext_v)
            pltpu.sync_copy(ones_v, acc.at[ext_v.at[1, :]], add=True)

        @pl.when(jnp.logical_and(c == 1, s == 1))
        def _():
            pltpu.sync_copy(tails[t], ext_v)
            pltpu.sync_copy(ones_v, acc.at[ext_v.at[1, :]], add=True)
    plsc.subcore_barrier()

    for t in range(2):
        pltpu.sync_copy(accs[t].at[pl.ds(s * _SEG, _SEG)], stage_v)
        pltpu.sync_copy(
            stage_v,
            out_hbm.at[pl.ds((2 * t + c) * _NPAD + s * _SEG, _SEG)])


_sc_count = pl.kernel(
    _sc_count_body,
    out_type=jax.ShapeDtypeStruct((4 * _NPAD,), jnp.float32),
    mesh=plsc.VectorSubcoreMesh(core_axis_name="c", subcore_axis_name="s"),
    scratch_types=[
        pltpu.VMEM((2, _SLAB), jnp.int32),
        pltpu.VMEM((2, 128), jnp.int32),
        pltpu.VMEM((128,), jnp.float32),
        pltpu.VMEM((_SEG,), jnp.float32),
        pltpu.SemaphoreType.DMA,
        pltpu.VMEM_SHARED((_NPAD,), jnp.float32),
        pltpu.VMEM_SHARED((_NPAD,), jnp.float32),
    ],
)


def _tc1_body(x_ref, w_ref, b_ref, kw_ref, kb_ref, q_ref, lw_ref,
              p00_ref, p01_ref, p10_ref, p11_ref, ytu_ref, stats_ref, g_sc):
    i = pl.program_id(0)
    live = (i > 0).astype(jnp.float32)
    gp = jnp.where(i > 0, g_sc[...], 0.0)
    g = jnp.maximum(
        jnp.dot(x_ref[...], w_ref[...], preferred_element_type=jnp.float32)
        + b_ref[...], 0.0)
    ytu_ref[...] = lax.dot_general(
        lw_ref[...], g, (((0,), (1,)), ((), ())),
        preferred_element_type=jnp.float32)
    g_sc[...] = g

    t = jnp.tanh(
        jnp.dot(gp, kw_ref[...], preferred_element_type=jnp.float32)
        + kb_ref[...])
    row = lax.broadcasted_iota(jnp.int32, (_BLK, HID), 0) + (i - 1) * _BLK
    t = jnp.where(row < N_MOVIE, t, 0.0)
    lane = lax.broadcasted_iota(jnp.int32, (1, _BLK), 1) + (i - 1) * _BLK
    valid = lane < N_MOVIE
    cnt0 = (p00_ref[...] + p01_ref[...]).reshape(1, _BLK)
    cnt1 = (p10_ref[...] + p11_ref[...]).reshape(1, _BLK)
    m0 = jnp.logical_and(cnt0 > 0.0, valid).astype(jnp.float32) * live
    m1 = jnp.logical_and(cnt1 > 0.0, valid).astype(jnp.float32) * live
    s0 = jnp.dot(m0, t, preferred_element_type=jnp.float32)
    s1 = jnp.dot(m1, t, preferred_element_type=jnp.float32)
    n0 = jnp.full((1, HID), jnp.sum(m0))
    n1 = jnp.full((1, HID), jnp.sum(m1))
    z = jnp.zeros((4, HID), jnp.float32)
    upd = jnp.concatenate([s0, s1, n0, n1, z], axis=0)

    @pl.when(i == 0)
    def _():
        stats_ref[...] = jnp.zeros((8, HID), jnp.float32)

    stats_ref[...] = stats_ref[...] + upd

    @pl.when(i == _NBLK)
    def _():
        tkb = jnp.tanh(kb_ref[...])
        n = jnp.float32(N_MOVIE)
        qv = q_ref[...]
        mean0 = (stats_ref[0:1, :] + (n - stats_ref[2, 0]) * tkb) / n
        mean1 = (stats_ref[1:2, :] + (n - stats_ref[3, 0]) * tkb) / n
        sc0 = jnp.sum(qv * mean0)
        sc1 = jnp.sum(qv * mean1)
        mx = jnp.maximum(sc0, sc1)
        e0 = jnp.exp(sc0 - mx)
        e1 = jnp.exp(sc1 - mx)
        stats_ref[4:5, :] = jnp.full((1, HID), e0 / (e0 + e1))
        stats_ref[5:6, :] = jnp.full((1, HID), e1 / (e0 + e1))


_tc1 = pl.pallas_call(
    _tc1_body,
    grid=(_NBLK + 1,),
    in_specs=[
        pl.BlockSpec((_BLK, HID),
                     lambda i: (jnp.minimum(i, _NBLK - 1), 0)),
        pl.BlockSpec((HID, HID), lambda i: (0, 0)),
        pl.BlockSpec((1, HID), lambda i: (0, 0)),
        pl.BlockSpec((HID, HID), lambda i: (0, 0)),
        pl.BlockSpec((1, HID), lambda i: (0, 0)),
        pl.BlockSpec((1, HID), lambda i: (0, 0)),
        pl.BlockSpec((HID, OUT_CH), lambda i: (0, 0)),
        pl.BlockSpec((_BLK,),
                     lambda i: (jnp.maximum(i, 1) - 1,)),
        pl.BlockSpec((_BLK,),
                     lambda i: (jnp.maximum(i, 1) - 1 + _NPAD // _BLK,)),
        pl.BlockSpec((_BLK,),
                     lambda i: (jnp.maximum(i, 1) - 1 + 2 * (_NPAD // _BLK),)),
        pl.BlockSpec((_BLK,),
                     lambda i: (jnp.maximum(i, 1) - 1 + 3 * (_NPAD // _BLK),)),
    ],
    out_specs=[
        pl.BlockSpec((OUT_CH, _BLK),
                     lambda i: (0, jnp.minimum(i, _NBLK - 1))),
        pl.BlockSpec((8, HID), lambda i: (0, 0)),
    ],
    out_shape=[
        jax.ShapeDtypeStruct((OUT_CH, N_MOVIE), jnp.float32),
        jax.ShapeDtypeStruct((8, HID), jnp.float32),
    ],
    scratch_shapes=[pltpu.VMEM((_BLK, HID), jnp.float32)],
)


def _tc2_body(ytu_ref, p00_ref, p01_ref, p10_ref, p11_ref, stats_ref,
              lbt_ref, yt_ref):
    a0 = stats_ref[4, 0]
    a1 = stats_ref[5, 0]
    cnt0 = (p00_ref[...] + p01_ref[...]).reshape(1, _BLK_C)
    cnt1 = (p10_ref[...] + p11_ref[...]).reshape(1, _BLK_C)
    m0 = (cnt0 > 0.0).astype(jnp.float32)
    m1 = (cnt1 > 0.0).astype(jnp.float32)
    coeff = a0 * m0 + a1 * m1
    yt_ref[...] = ytu_ref[...] * coeff + lbt_ref[...]


_tc2 = pl.pallas_call(
    _tc2_body,
    grid=(_NBLK_C,),
    in_specs=[
        pl.BlockSpec((OUT_CH, _BLK_C), lambda i: (0, i)),
        pl.BlockSpec((_BLK_C,), lambda i: (i,)),
        pl.BlockSpec((_BLK_C,), lambda i: (i + _NPAD // _BLK_C,)),
        pl.BlockSpec((_BLK_C,), lambda i: (i + 2 * (_NPAD // _BLK_C),)),
        pl.BlockSpec((_BLK_C,), lambda i: (i + 3 * (_NPAD // _BLK_C),)),
        pl.BlockSpec((8, HID), lambda i: (0, 0)),
        pl.BlockSpec((OUT_CH, 1), lambda i: (0, 0)),
    ],
    out_specs=pl.BlockSpec((OUT_CH, _BLK_C), lambda i: (0, i)),
    out_shape=jax.ShapeDtypeStruct((OUT_CH, N_MOVIE), jnp.float32),
)


@jax.jit
def _run(x_movie, ei0, ei1, proj_W, proj_b, k_W, k_b, q, lin_W, lin_b):
    tail0 = jnp.full((2, 128), N_MOVIE, jnp.int32).at[:, :32].set(
        ei0[:, 499968:])
    tail1 = jnp.full((2, 128), N_MOVIE, jnp.int32).at[:, :32].set(
        ei1[:, 499968:])
    counts = _sc_count(ei0, ei1, tail0, tail1)
    ytu, stats = _tc1(x_movie, proj_W, proj_b.reshape(1, HID),
                      k_W, k_b.reshape(1, HID), q, lin_W,
                      counts, counts, counts, counts)
    yt = _tc2(ytu, counts, counts, counts, counts, stats,
              lin_b.reshape(OUT_CH, 1))
    return yt.T


def kernel(x_movie, x_director, x_actor, edge_index_movie__to__director,
           edge_index_director__to__movie, edge_index_movie__to__actor,
           edge_index_actor__to__movie, proj_W_movie, proj_b_movie,
           proj_W_director, proj_b_director, proj_W_actor, proj_b_actor,
           lin_src_movie__to__director, lin_dst_movie__to__director,
           lin_src_director__to__movie, lin_dst_director__to__movie,
           lin_src_movie__to__actor, lin_dst_movie__to__actor,
           lin_src_actor__to__movie, lin_dst_actor__to__movie,
           k_W, k_b, q, lin_W, lin_b):
    return _run(x_movie, edge_index_director__to__movie,
                edge_index_actor__to__movie, proj_W_movie, proj_b_movie,
                k_W, k_b, q, lin_W, lin_b)


# speedup vs baseline: 1.1049x; 1.1049x over previous
import jax
import jax.numpy as jnp
from jax import lax
from jax.experimental import pallas as pl
from jax.experimental.pallas import tpu as pltpu
from jax.experimental.pallas import tpu_sc as plsc

N_MOVIE = 50000
E = 500000
HID = 128
OUT_CH = 16

_BLK = 4096
_NBLK = 13
_BLK_C = 16384
_NBLK_C = 4
_NSLICE = _NBLK * _BLK

_SLAB = 15616
_NCHS = _SLAB // 128
_GRP = 16
_C1_BASE = 249984
_SEG = 4096
_NPAD = 16 * _SEG


def _sc_count_body(e0_hbm, e1_hbm, t0_hbm, t1_hbm, out_hbm, idx_v, ext_v,
                   ones_v, stage_v, sem, acc0_sh, acc1_sh):
    c = lax.axis_index("c")
    s = lax.axis_index("s")
    edges = (e0_hbm, e1_hbm)
    tails = (t0_hbm, t1_hbm)
    accs = (acc0_sh, acc1_sh)

    def fill_ones(j, _):
        ones_v[pl.ds(j * 16, 16)] = jnp.ones((16,), jnp.float32)
        return _
    lax.fori_loop(0, 128 // 16, fill_ones, None)

    def fill_zeros(j, _):
        stage_v[pl.ds(j * 16, 16)] = jnp.zeros((16,), jnp.float32)
        return _
    lax.fori_loop(0, _SEG // 16, fill_zeros, None)

    for t in range(2):
        pltpu.sync_copy(stage_v, accs[t].at[pl.ds(s * _SEG, _SEG)])
    plsc.subcore_barrier()

    base = c * _C1_BASE + s * _SLAB
    for t in range(2):
        acc = accs[t]
        pltpu.sync_copy(edges[t].at[:, pl.ds(base, _SLAB)], idx_v)

        def scatter_grp(jo, _):
            cps = [
                pltpu.async_copy(
                    ones_v,
                    acc.at[idx_v.at[1, pl.ds((jo * _GRP + b) * 128, 128)]],
                    sem, add=True)
                for b in range(_GRP)
            ]
            for cp in cps:
                cp.wait()
            return _
        lax.fori_loop(0, _NCHS // _GRP, scatter_grp, None)
        for j in range(_NCHS - _NCHS % _GRP, _NCHS):
            pltpu.sync_copy(ones_v,
                            acc.at[idx_v.at[1, pl.ds(j * 128, 128)]],
                            add=True)

        @pl.when(jnp.logical_and(c == 0, s == 0))
        def _():
            pltpu.sync_copy(edges[t].at[:, pl.ds(249856, 128)], ext_v)
            pltpu.sync_copy(ones_v, acc.at[ext_v.at[1, :]], add=True)

        @pl.when(jnp.logical_and(c == 1, s == 0))
        def _():
            pltpu.sync_copy(edges[t].at[:, pl.ds(499840, 128)], ext_v)
            pltpu.sync_copy(ones_v, acc.at[ext_v.at[1, :]], add=True)

        @pl.when(jnp.logical_and(c == 1, s == 1))
        def _():
            pltpu.sync_copy(tails[t], ext_v)
            pltpu.sync_copy(ones_v, acc.at[ext_v.at[1, :]], add=True)
    plsc.subcore_barrier()

    for t in range(2):
        pltpu.sync_copy(accs[t].at[pl.ds(s * _SEG, _SEG)], stage_v)
        pltpu.sync_copy(
            stage_v,
            out_hbm.at[pl.ds((2 * t + c) * _NPAD + s * _SEG, _SEG)])


_sc_count = pl.kernel(
    _sc_count_body,
    out_type=jax.ShapeDtypeStruct((4 * _NPAD,), jnp.float32),
    mesh=plsc.VectorSubcoreMesh(core_axis_name="c", subcore_axis_name="s"),
    scratch_types=[
        pltpu.VMEM((2, _SLAB), jnp.int32),
        pltpu.VMEM((2, 128), jnp.int32),
        pltpu.VMEM((128,), jnp.float32),
        pltpu.VMEM((_SEG,), jnp.float32),
        pltpu.SemaphoreType.DMA,
        pltpu.VMEM_SHARED((_NPAD,), jnp.float32),
        pltpu.VMEM_SHARED((_NPAD,), jnp.float32),
    ],
)


def _tc1_body(x_ref, w_ref, b_ref, kw_ref, kb_ref, q_ref, lw_ref,
              p00_ref, p01_ref, p10_ref, p11_ref, ytu_ref, stats_ref):
    i = pl.program_id(0)
    g = jnp.maximum(
        jnp.dot(x_ref[...], w_ref[...], preferred_element_type=jnp.float32)
        + b_ref[...], 0.0)
    ytu_ref[...] = lax.dot_general(
        lw_ref[...], g, (((0,), (1,)), ((), ())),
        preferred_element_type=jnp.float32)
    t = jnp.tanh(
        jnp.dot(g, kw_ref[...], preferred_element_type=jnp.float32)
        + kb_ref[...])
    row = lax.broadcasted_iota(jnp.int32, (_BLK, HID), 0) + i * _BLK
    t = jnp.where(row < N_MOVIE, t, 0.0)
    lane = lax.broadcasted_iota(jnp.int32, (1, _BLK), 1) + i * _BLK
    valid = lane < N_MOVIE
    cnt0 = (p00_ref[...] + p01_ref[...]).reshape(1, _BLK)
    cnt1 = (p10_ref[...] + p11_ref[...]).reshape(1, _BLK)
    m0 = jnp.logical_and(cnt0 > 0.0, valid).astype(jnp.float32)
    m1 = jnp.logical_and(cnt1 > 0.0, valid).astype(jnp.float32)
    s0 = jnp.dot(m0, t, preferred_element_type=jnp.float32)
    s1 = jnp.dot(m1, t, preferred_element_type=jnp.float32)
    n0 = jnp.full((1, HID), jnp.sum(m0))
    n1 = jnp.full((1, HID), jnp.sum(m1))
    z = jnp.zeros((4, HID), jnp.float32)
    upd = jnp.concatenate([s0, s1, n0, n1, z], axis=0)

    @pl.when(i == 0)
    def _():
        stats_ref[...] = upd

    @pl.when(i > 0)
    def _():
        stats_ref[...] = stats_ref[...] + upd

    @pl.when(i == _NBLK - 1)
    def _():
        tkb = jnp.tanh(kb_ref[...])
        n = jnp.float32(N_MOVIE)
        qv = q_ref[...]
        mean0 = (stats_ref[0:1, :] + (n - stats_ref[2, 0]) * tkb) / n
        mean1 = (stats_ref[1:2, :] + (n - stats_ref[3, 0]) * tkb) / n
        sc0 = jnp.sum(qv * mean0)
        sc1 = jnp.sum(qv * mean1)
        mx = jnp.maximum(sc0, sc1)
        e0 = jnp.exp(sc0 - mx)
        e1 = jnp.exp(sc1 - mx)
        stats_ref[4:5, :] = jnp.full((1, HID), e0 / (e0 + e1))
        stats_ref[5:6, :] = jnp.full((1, HID), e1 / (e0 + e1))


_tc1 = pl.pallas_call(
    _tc1_body,
    grid=(_NBLK,),
    in_specs=[
        pl.BlockSpec((_BLK, HID), lambda i: (i, 0)),
        pl.BlockSpec((HID, HID), lambda i: (0, 0)),
        pl.BlockSpec((1, HID), lambda i: (0, 0)),
        pl.BlockSpec((HID, HID), lambda i: (0, 0)),
        pl.BlockSpec((1, HID), lambda i: (0, 0)),
        pl.BlockSpec((1, HID), lambda i: (0, 0)),
        pl.BlockSpec((HID, OUT_CH), lambda i: (0, 0)),
        pl.BlockSpec((_BLK,), lambda i: (i,)),
        pl.BlockSpec((_BLK,), lambda i: (i + _NPAD // _BLK,)),
        pl.BlockSpec((_BLK,), lambda i: (i + 2 * (_NPAD // _BLK),)),
        pl.BlockSpec((_BLK,), lambda i: (i + 3 * (_NPAD // _BLK),)),
    ],
    out_specs=[
        pl.BlockSpec((OUT_CH, _BLK), lambda i: (0, i)),
        pl.BlockSpec((8, HID), lambda i: (0, 0)),
    ],
    out_shape=[
        jax.ShapeDtypeStruct((OUT_CH, N_MOVIE), jnp.float32),
        jax.ShapeDtypeStruct((8, HID), jnp.float32),
    ],
)


def _tc2_body(ytu_ref, p00_ref, p01_ref, p10_ref, p11_ref, stats_ref,
              lbt_ref, yt_ref):
    a0 = stats_ref[4, 0]
    a1 = stats_ref[5, 0]
    cnt0 = (p00_ref[...] + p01_ref[...]).reshape(1, _BLK_C)
    cnt1 = (p10_ref[...] + p11_ref[...]).reshape(1, _BLK_C)
    m0 = (cnt0 > 0.0).astype(jnp.float32)
    m1 = (cnt1 > 0.0).astype(jnp.float32)
    coeff = a0 * m0 + a1 * m1
    yt_ref[...] = ytu_ref[...] * coeff + lbt_ref[...]


_tc2 = pl.pallas_call(
    _tc2_body,
    grid=(_NBLK_C,),
    in_specs=[
        pl.BlockSpec((OUT_CH, _BLK_C), lambda i: (0, i)),
        pl.BlockSpec((_BLK_C,), lambda i: (i,)),
        pl.BlockSpec((_BLK_C,), lambda i: (i + _NPAD // _BLK_C,)),
        pl.BlockSpec((_BLK_C,), lambda i: (i + 2 * (_NPAD // _BLK_C),)),
        pl.BlockSpec((_BLK_C,), lambda i: (i + 3 * (_NPAD // _BLK_C),)),
        pl.BlockSpec((8, HID), lambda i: (0, 0)),
        pl.BlockSpec((OUT_CH, 1), lambda i: (0, 0)),
    ],
    out_specs=pl.BlockSpec((OUT_CH, _BLK_C), lambda i: (0, i)),
    out_shape=jax.ShapeDtypeStruct((OUT_CH, N_MOVIE), jnp.float32),
)


@jax.jit
def _run(x_movie, ei0, ei1, proj_W, proj_b, k_W, k_b, q, lin_W, lin_b):
    tails = jnp.full((2, 2, 128), N_MOVIE, jnp.int32).at[:, :, :32].set(
        jnp.stack([ei0[:, 499968:], ei1[:, 499968:]]))
    counts = _sc_count(ei0, ei1, tails[0], tails[1])
    ytu, stats = _tc1(x_movie, proj_W, proj_b.reshape(1, HID),
                      k_W, k_b.reshape(1, HID), q, lin_W,
                      counts, counts, counts, counts)
    yt = _tc2(ytu, counts, counts, counts, counts, stats,
              lin_b.reshape(OUT_CH, 1))
    return yt.T


def kernel(x_movie, x_director, x_actor, edge_index_movie__to__director,
           edge_index_director__to__movie, edge_index_movie__to__actor,
           edge_index_actor__to__movie, proj_W_movie, proj_b_movie,
           proj_W_director, proj_b_director, proj_W_actor, proj_b_actor,
           lin_src_movie__to__director, lin_dst_movie__to__director,
           lin_src_director__to__movie, lin_dst_director__to__movie,
           lin_src_movie__to__actor, lin_dst_movie__to__actor,
           lin_src_actor__to__movie, lin_dst_actor__to__movie,
           k_W, k_b, q, lin_W, lin_b):
    return _run(x_movie, edge_index_director__to__movie,
                edge_index_actor__to__movie, proj_W_movie, proj_b_movie,
                k_W, k_b, q, lin_W, lin_b)


# speedup vs baseline: 1.1075x; 1.0023x over previous
import jax
import jax.numpy as jnp
from jax import lax
from jax.experimental import pallas as pl
from jax.experimental.pallas import tpu as pltpu
from jax.experimental.pallas import tpu_sc as plsc

N_MOVIE = 50000
E = 500000
HID = 128
OUT_CH = 16

_BLK = 8192
_NBLK = 7
_BLK_C = 16384
_NBLK_C = 4
_NSLICE = _NBLK * _BLK

_SLAB = 15616
_NCHS = _SLAB // 128
_GRP = 16
_C1_BASE = 249984
_SEG = 4096
_NPAD = 16 * _SEG


def _sc_count_body(e0_hbm, e1_hbm, t0_hbm, t1_hbm, out_hbm, idx_v, ext_v,
                   ones_v, stage_v, sem, acc0_sh, acc1_sh):
    c = lax.axis_index("c")
    s = lax.axis_index("s")
    edges = (e0_hbm, e1_hbm)
    tails = (t0_hbm, t1_hbm)
    accs = (acc0_sh, acc1_sh)

    def fill_ones(j, _):
        ones_v[pl.ds(j * 16, 16)] = jnp.ones((16,), jnp.float32)
        return _
    lax.fori_loop(0, 128 // 16, fill_ones, None)

    def fill_zeros(j, _):
        stage_v[pl.ds(j * 16, 16)] = jnp.zeros((16,), jnp.float32)
        return _
    lax.fori_loop(0, _SEG // 16, fill_zeros, None)

    for t in range(2):
        pltpu.sync_copy(stage_v, accs[t].at[pl.ds(s * _SEG, _SEG)])
    plsc.subcore_barrier()

    base = c * _C1_BASE + s * _SLAB
    for t in range(2):
        acc = accs[t]
        pltpu.sync_copy(edges[t].at[:, pl.ds(base, _SLAB)], idx_v)

        def scatter_grp(jo, _):
            cps = [
                pltpu.async_copy(
                    ones_v,
                    acc.at[idx_v.at[1, pl.ds((jo * _GRP + b) * 128, 128)]],
                    sem, add=True)
                for b in range(_GRP)
            ]
            for cp in cps:
                cp.wait()
            return _
        lax.fori_loop(0, _NCHS // _GRP, scatter_grp, None)
        for j in range(_NCHS - _NCHS % _GRP, _NCHS):
            pltpu.sync_copy(ones_v,
                            acc.at[idx_v.at[1, pl.ds(j * 128, 128)]],
                            add=True)

        @pl.when(jnp.logical_and(c == 0, s == 0))
        def _():
            pltpu.sync_copy(edges[t].at[:, pl.ds(249856, 128)], ext_v)
            pltpu.sync_copy(ones_v, acc.at[ext_v.at[1, :]], add=True)

        @pl.when(jnp.logical_and(c == 1, s == 0))
        def _():
            pltpu.sync_copy(edges[t].at[:, pl.ds(499840, 128)], ext_v)
            pltpu.sync_copy(ones_v, acc.at[ext_v.at[1, :]], add=True)

        @pl.when(jnp.logical_and(c == 1, s == 1))
        def _():
            pltpu.sync_copy(tails[t], ext_v)
            pltpu.sync_copy(ones_v, acc.at[ext_v.at[1, :]], add=True)
    plsc.subcore_barrier()

    for t in range(2):
        pltpu.sync_copy(accs[t].at[pl.ds(s * _SEG, _SEG)], stage_v)
        pltpu.sync_copy(
            stage_v,
            out_hbm.at[pl.ds((2 * t + c) * _NPAD + s * _SEG, _SEG)])


_sc_count = pl.kernel(
    _sc_count_body,
    out_type=jax.ShapeDtypeStruct((4 * _NPAD,), jnp.float32),
    mesh=plsc.VectorSubcoreMesh(core_axis_name="c", subcore_axis_name="s"),
    scratch_types=[
        pltpu.VMEM((2, _SLAB), jnp.int32),
        pltpu.VMEM((2, 128), jnp.int32),
        pltpu.VMEM((128,), jnp.float32),
        pltpu.VMEM((_SEG,), jnp.float32),
        pltpu.SemaphoreType.DMA,
        pltpu.VMEM_SHARED((_NPAD,), jnp.float32),
        pltpu.VMEM_SHARED((_NPAD,), jnp.float32),
    ],
)


def _tc1_body(x_ref, w_ref, b_ref, kw_ref, kb_ref, q_ref, lw_ref,
              p00_ref, p01_ref, p10_ref, p11_ref, ytu_ref, stats_ref):
    i = pl.program_id(0)
    g = jnp.maximum(
        jnp.dot(x_ref[...], w_ref[...], preferred_element_type=jnp.float32)
        + b_ref[...], 0.0)
    ytu_ref[...] = lax.dot_general(
        lw_ref[...], g, (((0,), (1,)), ((), ())),
        preferred_element_type=jnp.float32)
    t = jnp.tanh(
        jnp.dot(g, kw_ref[...], preferred_element_type=jnp.float32)
        + kb_ref[...])
    row = lax.broadcasted_iota(jnp.int32, (_BLK, HID), 0) + i * _BLK
    t = jnp.where(row < N_MOVIE, t, 0.0)
    lane = lax.broadcasted_iota(jnp.int32, (1, _BLK), 1) + i * _BLK
    valid = lane < N_MOVIE
    cnt0 = (p00_ref[...] + p01_ref[...]).reshape(1, _BLK)
    cnt1 = (p10_ref[...] + p11_ref[...]).reshape(1, _BLK)
    m0 = jnp.logical_and(cnt0 > 0.0, valid).astype(jnp.float32)
    m1 = jnp.logical_and(cnt1 > 0.0, valid).astype(jnp.float32)
    s0 = jnp.dot(m0, t, preferred_element_type=jnp.float32)
    s1 = jnp.dot(m1, t, preferred_element_type=jnp.float32)
    n0 = jnp.full((1, HID), jnp.sum(m0))
    n1 = jnp.full((1, HID), jnp.sum(m1))
    z = jnp.zeros((4, HID), jnp.float32)
    upd = jnp.concatenate([s0, s1, n0, n1, z], axis=0)

    @pl.when(i == 0)
    def _():
        stats_ref[...] = upd

    @pl.when(i > 0)
    def _():
        stats_ref[...] = stats_ref[...] + upd

    @pl.when(i == _NBLK - 1)
    def _():
        tkb = jnp.tanh(kb_ref[...])
        n = jnp.float32(N_MOVIE)
        qv = q_ref[...]
        mean0 = (stats_ref[0:1, :] + (n - stats_ref[2, 0]) * tkb) / n
        mean1 = (stats_ref[1:2, :] + (n - stats_ref[3, 0]) * tkb) / n
        sc0 = jnp.sum(qv * mean0)
        sc1 = jnp.sum(qv * mean1)
        mx = jnp.maximum(sc0, sc1)
        e0 = jnp.exp(sc0 - mx)
        e1 = jnp.exp(sc1 - mx)
        stats_ref[4:5, :] = jnp.full((1, HID), e0 / (e0 + e1))
        stats_ref[5:6, :] = jnp.full((1, HID), e1 / (e0 + e1))


_tc1 = pl.pallas_call(
    _tc1_body,
    grid=(_NBLK,),
    in_specs=[
        pl.BlockSpec((_BLK, HID), lambda i: (i, 0)),
        pl.BlockSpec((HID, HID), lambda i: (0, 0)),
        pl.BlockSpec((1, HID), lambda i: (0, 0)),
        pl.BlockSpec((HID, HID), lambda i: (0, 0)),
        pl.BlockSpec((1, HID), lambda i: (0, 0)),
        pl.BlockSpec((1, HID), lambda i: (0, 0)),
        pl.BlockSpec((HID, OUT_CH), lambda i: (0, 0)),
        pl.BlockSpec((_BLK,), lambda i: (i,)),
        pl.BlockSpec((_BLK,), lambda i: (i + _NPAD // _BLK,)),
        pl.BlockSpec((_BLK,), lambda i: (i + 2 * (_NPAD // _BLK),)),
        pl.BlockSpec((_BLK,), lambda i: (i + 3 * (_NPAD // _BLK),)),
    ],
    out_specs=[
        pl.BlockSpec((OUT_CH, _BLK), lambda i: (0, i)),
        pl.BlockSpec((8, HID), lambda i: (0, 0)),
    ],
    out_shape=[
        jax.ShapeDtypeStruct((OUT_CH, N_MOVIE), jnp.float32),
        jax.ShapeDtypeStruct((8, HID), jnp.float32),
    ],
)


def _tc2_body(ytu_ref, p00_ref, p01_ref, p10_ref, p11_ref, stats_ref,
              lbt_ref, yt_ref):
    a0 = stats_ref[4, 0]
    a1 = stats_ref[5, 0]
    cnt0 = (p00_ref[...] + p01_ref[...]).reshape(1, _BLK_C)
    cnt1 = (p10_ref[...] + p11_ref[...]).reshape(1, _BLK_C)
    m0 = (cnt0 > 0.0).astype(jnp.float32)
    m1 = (cnt1 > 0.0).astype(jnp.float32)
    coeff = a0 * m0 + a1 * m1
    yt_ref[...] = ytu_ref[...] * coeff + lbt_ref[...]


_tc2 = pl.pallas_call(
    _tc2_body,
    grid=(_NBLK_C,),
    in_specs=[
        pl.BlockSpec((OUT_CH, _BLK_C), lambda i: (0, i)),
        pl.BlockSpec((_BLK_C,), lambda i: (i,)),
        pl.BlockSpec((_BLK_C,), lambda i: (i + _NPAD // _BLK_C,)),
        pl.BlockSpec((_BLK_C,), lambda i: (i + 2 * (_NPAD // _BLK_C),)),
        pl.BlockSpec((_BLK_C,), lambda i: (i + 3 * (_NPAD // _BLK_C),)),
        pl.BlockSpec((8, HID), lambda i: (0, 0)),
        pl.BlockSpec((OUT_CH, 1), lambda i: (0, 0)),
    ],
    out_specs=pl.BlockSpec((OUT_CH, _BLK_C), lambda i: (0, i)),
    out_shape=jax.ShapeDtypeStruct((OUT_CH, N_MOVIE), jnp.float32),
)


@jax.jit
def _run(x_movie, ei0, ei1, proj_W, proj_b, k_W, k_b, q, lin_W, lin_b):
    tails = jnp.full((2, 2, 128), N_MOVIE, jnp.int32).at[:, :, :32].set(
        jnp.stack([ei0[:, 499968:], ei1[:, 499968:]]))
    counts = _sc_count(ei0, ei1, tails[0], tails[1])
    ytu, stats = _tc1(x_movie, proj_W, proj_b.reshape(1, HID),
                      k_W, k_b.reshape(1, HID), q, lin_W,
                      counts, counts, counts, counts)
    yt = _tc2(ytu, counts, counts, counts, counts, stats,
              lin_b.reshape(OUT_CH, 1))
    return yt.T


def kernel(x_movie, x_director, x_actor, edge_index_movie__to__director,
           edge_index_director__to__movie, edge_index_movie__to__actor,
           edge_index_actor__to__movie, proj_W_movie, proj_b_movie,
           proj_W_director, proj_b_director, proj_W_actor, proj_b_actor,
           lin_src_movie__to__director, lin_dst_movie__to__director,
           lin_src_director__to__movie, lin_dst_director__to__movie,
           lin_src_movie__to__actor, lin_dst_movie__to__actor,
           lin_src_actor__to__movie, lin_dst_actor__to__movie,
           k_W, k_b, q, lin_W, lin_b):
    return _run(x_movie, edge_index_director__to__movie,
                edge_index_actor__to__movie, proj_W_movie, proj_b_movie,
                k_W, k_b, q, lin_W, lin_b)
